# baseline (speedup 1.0000x reference)
import jax
import jax.numpy as jnp
from jax.experimental import pallas as pl
from jax.experimental.pallas import tpu as pltpu

_HIDDEN = 2048
_EXPERTS = 64
_BLK = 1024


def _router_block(x_ref, w_hbm, o_ref, w_vmem, w_sem):
    @pl.when(pl.program_id(0) == 0)
    def _():
        cp = pltpu.make_async_copy(w_hbm, w_vmem, w_sem)
        cp.start()
        cp.wait()

    x = x_ref[...].astype(jnp.bfloat16)
    w = w_vmem[...].astype(jnp.bfloat16)
    o_ref[...] = jax.lax.dot_general(
        x, w,
        dimension_numbers=(((1,), (1,)), ((), ())),
        preferred_element_type=jnp.float32,
    )


def kernel(hidden_states, weight):
    hs = hidden_states.reshape(-1, _HIDDEN)
    n = hs.shape[0]
    return pl.pallas_call(
        _router_block,
        grid=(n // _BLK,),
        in_specs=[
            pl.BlockSpec((_BLK, _HIDDEN), lambda i: (i, 0)),
            pl.BlockSpec(memory_space=pltpu.HBM),
        ],
        out_specs=pl.BlockSpec((_BLK, _EXPERTS), lambda i: (i, 0)),
        out_shape=jax.ShapeDtypeStruct((n, _EXPERTS), jnp.float32),
        scratch_shapes=[
            pltpu.VMEM((_EXPERTS, _HIDDEN), jnp.float32),
            pltpu.SemaphoreType.DMA,
        ],
        compiler_params=pltpu.CompilerParams(
            dimension_semantics=(pltpu.ARBITRARY,),
            vmem_limit_bytes=100 * 1024 * 1024,
        ),
    )(hs, weight)


# speedup vs baseline: 1.0020x; 1.0020x over previous
import jax
import jax.numpy as jnp
from jax.experimental import pallas as pl
from jax.experimental.pallas import tpu as pltpu

_HIDDEN = 2048
_EXPERTS = 64
_BLK = 1024


def _router_block(x_ref, w_hbm, o_ref, w_vmem, w_sem):
    @pl.when(pl.program_id(0) == 0)
    def _():
        cp = pltpu.make_async_copy(w_hbm, w_vmem, w_sem)
        cp.start()
        cp.wait()

    x = x_ref[...].astype(jnp.bfloat16)
    w = w_vmem[...].astype(jnp.bfloat16)
    o_ref[...] = jax.lax.dot_general(
        x, w,
        dimension_numbers=(((1,), (1,)), ((), ())),
        preferred_element_type=jnp.float32,
    )


def kernel(hidden_states, weight):
    hs = hidden_states.reshape(-1, _HIDDEN)
    n = hs.shape[0]
    return pl.pallas_call(
        _router_block,
        grid=(n // _BLK,),
        in_specs=[
            pl.BlockSpec((_BLK, _HIDDEN), lambda i: (i, 0)),
            pl.BlockSpec(memory_space=pltpu.HBM),
        ],
        out_specs=pl.BlockSpec((_BLK, _EXPERTS), lambda i: (i, 0)),
        out_shape=jax.ShapeDtypeStruct((n, _EXPERTS), jnp.float32),
        scratch_shapes=[
            pltpu.VMEM((_EXPERTS, _HIDDEN), jnp.float32),
            pltpu.SemaphoreType.DMA,
        ],
        compiler_params=pltpu.CompilerParams(
            dimension_semantics=(pltpu.PARALLEL,),
            vmem_limit_bytes=100 * 1024 * 1024,
        ),
    )(hs, weight)


# speedup vs baseline: 1.0127x; 1.0107x over previous
import jax
import jax.numpy as jnp
from jax.experimental import pallas as pl
from jax.experimental.pallas import tpu as pltpu

_HIDDEN = 2048
_EXPERTS = 64
_BLK = 2048


def _router_block(x_ref, w_ref, o_ref):
    x = x_ref[...].astype(jnp.bfloat16)
    w = w_ref[...].astype(jnp.bfloat16)
    o_ref[...] = jax.lax.dot_general(
        x, w,
        dimension_numbers=(((1,), (1,)), ((), ())),
        preferred_element_type=jnp.float32,
    )


def kernel(hidden_states, weight):
    hs = hidden_states.reshape(-1, _HIDDEN)
    n = hs.shape[0]
    return pl.pallas_call(
        _router_block,
        grid=(n // _BLK,),
        in_specs=[
            pl.BlockSpec((_BLK, _HIDDEN), lambda i: (i, 0)),
            pl.BlockSpec((_EXPERTS, _HIDDEN), lambda i: (0, 0)),
        ],
        out_specs=pl.BlockSpec((_BLK, _EXPERTS), lambda i: (i, 0)),
        out_shape=jax.ShapeDtypeStruct((n, _EXPERTS), jnp.float32),
        compiler_params=pltpu.CompilerParams(
            dimension_semantics=(pltpu.PARALLEL,),
            vmem_limit_bytes=100 * 1024 * 1024,
        ),
    )(hs, weight)


# speedup vs baseline: 1.0333x; 1.0203x over previous
import jax
import jax.numpy as jnp
from jax.experimental import pallas as pl
from jax.experimental.pallas import tpu as pltpu

_HIDDEN = 2048
_EXPERTS = 64
_BLK = 1024


def _router_block(x_ref, w_ref, o_ref):
    i = pl.program_id(0)
    x = x_ref[...].astype(jnp.bfloat16)
    w = w_ref[...].astype(jnp.bfloat16)
    o_ref[pl.ds(i * _BLK, _BLK), :] = jax.lax.dot_general(
        x, w,
        dimension_numbers=(((1,), (1,)), ((), ())),
        preferred_element_type=jnp.float32,
    )


def kernel(hidden_states, weight):
    hs = hidden_states.reshape(-1, _HIDDEN)
    n = hs.shape[0]
    return pl.pallas_call(
        _router_block,
        grid=(n // _BLK,),
        in_specs=[
            pl.BlockSpec((_BLK, _HIDDEN), lambda i: (i, 0)),
            pl.BlockSpec((_EXPERTS, _HIDDEN), lambda i: (0, 0)),
        ],
        out_specs=pl.BlockSpec(memory_space=pltpu.VMEM),
        out_shape=jax.ShapeDtypeStruct((n, _EXPERTS), jnp.float32),
        compiler_params=pltpu.CompilerParams(
            dimension_semantics=(pltpu.ARBITRARY,),
            vmem_limit_bytes=100 * 1024 * 1024,
        ),
    )(hs, weight)


# speedup vs baseline: 1.0396x; 1.0060x over previous
import jax
import jax.numpy as jnp
from jax.experimental import pallas as pl
from jax.experimental.pallas import tpu as pltpu

_HIDDEN = 2048
_EXPERTS = 64
_BLK = 256
_NSPLIT = 4


def _router_block(*refs):
    xs = refs[:_NSPLIT]
    w_ref = refs[_NSPLIT]
    o_ref = refs[_NSPLIT + 1]
    w = w_ref[...].astype(jnp.bfloat16)
    dn = (((1,), (1,)), ((), ()))
    for s in range(_NSPLIT):
        x = xs[s][0].astype(jnp.bfloat16)
        o_ref[s] = jax.lax.dot_general(
            x, w, dn, preferred_element_type=jnp.float32)


def kernel(hidden_states, weight):
    hs = hidden_states.reshape(-1, _HIDDEN)
    n = hs.shape[0]
    rows = n // _NSPLIT
    hs3 = hs.reshape(_NSPLIT, rows, _HIDDEN)
    steps = rows // _BLK

    def x_spec(s):
        return pl.BlockSpec((1, _BLK, _HIDDEN), lambda i, s=s: (s, i, 0))

    out = pl.pallas_call(
        _router_block,
        grid=(steps,),
        in_specs=[x_spec(s) for s in range(_NSPLIT)] + [
            pl.BlockSpec((_EXPERTS, _HIDDEN), lambda i: (0, 0)),
        ],
        out_specs=pl.BlockSpec((_NSPLIT, _BLK, _EXPERTS), lambda i: (0, i, 0)),
        out_shape=jax.ShapeDtypeStruct((_NSPLIT, rows, _EXPERTS), jnp.float32),
        compiler_params=pltpu.CompilerParams(
            dimension_semantics=(pltpu.PARALLEL,),
            vmem_limit_bytes=100 * 1024 * 1024,
        ),
    )(*([hs3] * _NSPLIT), weight)
    return out.reshape(n, _EXPERTS)
